# baseline (speedup 1.0000x reference)
import functools

import jax
import jax.numpy as jnp
from jax.experimental import pallas as pl
from jax.experimental.pallas import tpu as pltpu


def _conv0_pool_kernel(pe_ref, po_ref, w_ref, b_ref, o_ref):
    _, rows, Wo, K = pe_ref.shape
    tp = o_ref.shape[1]
    acc_e = jnp.dot(pe_ref[0].reshape(rows * Wo, K), w_ref[...],
                    preferred_element_type=jnp.float32)
    acc_o = jnp.dot(po_ref[0].reshape(rows * Wo, K), w_ref[...],
                    preferred_element_type=jnp.float32)
    m = jnp.maximum(acc_e, acc_o).reshape(tp, 2, Wo, 64)
    v = jnp.maximum(m[:, 0], m[:, 1])
    out = jnp.maximum(v + b_ref[...].reshape(1, 1, 64), 0.0)
    o_ref[0] = out.astype(o_ref.dtype)


def _conv_fold_pool_kernel(x_ref, w_ref, b_ref, o_ref, *, cin, pool):
    _, H, W, _ = x_ref.shape
    co = o_ref.shape[-1]
    xp = jnp.pad(x_ref[0], ((1, 1), (1, 1), (0, 0)))
    acc = None
    for dy in range(3):
        rows = xp[dy:dy + H]
        patch = jnp.concatenate(
            [rows[:, 0:W], rows[:, 1:W + 1], rows[:, 2:W + 2]],
            axis=-1).reshape(H * W, 3 * cin)
        d = jnp.dot(patch, w_ref[dy], preferred_element_type=jnp.float32)
        acc = d if acc is None else acc + d
    y = acc.reshape(H, W, co)
    if pool:
        y4 = y.reshape(H // 2, 2, W, co)
        v = jnp.maximum(y4[:, 0], y4[:, 1])
        z = v.reshape(H // 2, W // 2, 2 * co)
        y = jnp.maximum(z[..., :co], z[..., co:])
    out = jnp.maximum(y + b_ref[...].reshape(1, 1, co), 0.0)
    o_ref[0] = out.astype(o_ref.dtype)


def _conv0_pool(xf, w0, b0):
    N, H, W, _ = xf.shape
    Wo = W // 2
    xp = jnp.pad(xf, ((0, 0), (1, 1), (1, 1), (0, 0)))
    xe = xp[:, :, 0::2, :]
    xo = xp[:, :, 1::2, :]
    halves = [xe, xo]
    taps_e = [halves[dx % 2][:, dy:dy + H, dx // 2:dx // 2 + Wo, :]
              for dy in range(3) for dx in range(3)]
    taps_o = [halves[(dx + 1) % 2][:, dy:dy + H,
                                   (dx + 1) // 2:(dx + 1) // 2 + Wo, :]
              for dy in range(3) for dx in range(3)]
    zpad = jnp.zeros((N, H, Wo, 5), jnp.bfloat16)
    pe = jnp.concatenate(taps_e + [zpad], axis=-1)
    po = jnp.concatenate(taps_o + [zpad], axis=-1)
    tp = 8
    return pl.pallas_call(
        _conv0_pool_kernel,
        out_shape=jax.ShapeDtypeStruct((N, H // 2, Wo, 64), jnp.bfloat16),
        grid=(N, H // (2 * tp)),
        in_specs=[
            pl.BlockSpec((1, 2 * tp, Wo, 32), lambda n, r: (n, r, 0, 0)),
            pl.BlockSpec((1, 2 * tp, Wo, 32), lambda n, r: (n, r, 0, 0)),
            pl.BlockSpec((32, 64), lambda n, r: (0, 0)),
            pl.BlockSpec((1, 64), lambda n, r: (0, 0)),
        ],
        out_specs=pl.BlockSpec((1, tp, Wo, 64), lambda n, r: (n, r, 0, 0)),
        compiler_params=pltpu.CompilerParams(
            dimension_semantics=("parallel", "parallel")),
    )(pe, po, w0, b0)


def _conv_layer(x, w, b, *, cin, cout, pool):
    N, H, W, _ = x.shape
    Ho, Wo = (H // 2, W // 2) if pool else (H, W)
    return pl.pallas_call(
        functools.partial(_conv_fold_pool_kernel, cin=cin, pool=pool),
        out_shape=jax.ShapeDtypeStruct((N, Ho, Wo, cout), jnp.bfloat16),
        grid=(N,),
        in_specs=[
            pl.BlockSpec((1, H, W, cin), lambda n: (n, 0, 0, 0)),
            pl.BlockSpec((3, 3 * cin, cout), lambda n: (0, 0, 0)),
            pl.BlockSpec((1, cout), lambda n: (0, 0)),
        ],
        out_specs=pl.BlockSpec((1, Ho, Wo, cout), lambda n: (n, 0, 0, 0)),
        compiler_params=pltpu.CompilerParams(
            dimension_semantics=("parallel",),
            vmem_limit_bytes=48 * 1024 * 1024),
    )(x, w, b)


def _fc_kernel(x_ref, w_ref, b_ref, o_ref, acc_ref, *, relu):
    k = pl.program_id(1)

    @pl.when(k == 0)
    def _init():
        acc_ref[...] = jnp.zeros_like(acc_ref)

    acc_ref[...] += jnp.dot(x_ref[...], w_ref[...],
                            preferred_element_type=jnp.float32)

    @pl.when(k == pl.num_programs(1) - 1)
    def _done():
        r = acc_ref[...] + b_ref[...]
        if relu:
            r = jnp.maximum(r, 0.0)
        o_ref[...] = r.astype(o_ref.dtype)


def _fc(x, w, b, *, relu, out_dtype):
    M, K = x.shape
    Kp, Np = w.shape
    tk = next(t for t in (4096, 3584, 2048, 1024, 512, 256, 128) if Kp % t == 0)
    tn = next(t for t in (512, 256, 128) if Np % t == 0)
    return pl.pallas_call(
        functools.partial(_fc_kernel, relu=relu),
        out_shape=jax.ShapeDtypeStruct((M, Np), out_dtype),
        grid=(Np // tn, Kp // tk),
        in_specs=[
            pl.BlockSpec((M, tk), lambda j, k: (0, k)),
            pl.BlockSpec((tk, tn), lambda j, k: (k, j)),
            pl.BlockSpec((1, tn), lambda j, k: (0, j)),
        ],
        out_specs=pl.BlockSpec((M, tn), lambda j, k: (0, j)),
        scratch_shapes=[pltpu.VMEM((M, tn), jnp.float32)],
        compiler_params=pltpu.CompilerParams(
            dimension_semantics=("parallel", "arbitrary")),
    )(x.astype(jnp.bfloat16), w, b)


def kernel(x_nchw, conv_w_0, conv_b_0, conv_w_1, conv_b_1, conv_w_2, conv_b_2,
           conv_w_3, conv_b_3, conv_w_4, conv_b_4, conv_w_5, conv_b_5,
           conv_w_6, conv_b_6, conv_w_7, conv_b_7,
           fc_w_0, fc_b_0, fc_w_1, fc_b_1, fc_w_2, fc_b_2):
    N = x_nchw.shape[0]
    xf = jnp.transpose(x_nchw.astype(jnp.bfloat16), (0, 2, 3, 1))
    h = _conv0_pool(xf, conv_w_0[:32, :64], conv_b_0[:, :64])
    w1 = conv_w_1.reshape(3, 3, 128, 128)[:, :, :64, :].reshape(3, 192, 128)
    h = _conv_layer(h, w1, conv_b_1, cin=64, cout=128, pool=True)
    h = _conv_layer(h, conv_w_2, conv_b_2, cin=128, cout=256, pool=False)
    h = _conv_layer(h, conv_w_3, conv_b_3, cin=256, cout=256, pool=True)
    h = _conv_layer(h, conv_w_4, conv_b_4, cin=256, cout=512, pool=False)
    h = _conv_layer(h, conv_w_5, conv_b_5, cin=512, cout=512, pool=True)
    h = _conv_layer(h, conv_w_6, conv_b_6, cin=512, cout=512, pool=False)
    h = _conv_layer(h, conv_w_7, conv_b_7, cin=512, cout=512, pool=True)
    f = h.reshape(N, -1)
    f = _fc(f, fc_w_0, fc_b_0, relu=True, out_dtype=jnp.bfloat16)
    f = _fc(f, fc_w_1, fc_b_1, relu=True, out_dtype=jnp.bfloat16)
    f = _fc(f, fc_w_2, fc_b_2, relu=False, out_dtype=jnp.float32)
    return f[:, :10]


# speedup vs baseline: 3.9949x; 3.9949x over previous
import functools

import jax
import jax.numpy as jnp
from jax.experimental import pallas as pl
from jax.experimental.pallas import tpu as pltpu


def _conv0_pool_kernel(pe_ref, po_ref, w_ref, b_ref, o_ref):
    _, rows, Wo, K = pe_ref.shape
    tp = o_ref.shape[1]
    acc_e = jnp.dot(pe_ref[0].reshape(rows * Wo, K), w_ref[...],
                    preferred_element_type=jnp.float32)
    acc_o = jnp.dot(po_ref[0].reshape(rows * Wo, K), w_ref[...],
                    preferred_element_type=jnp.float32)
    m = jnp.maximum(acc_e, acc_o).reshape(tp, 2, Wo, 64)
    v = jnp.maximum(m[:, 0], m[:, 1])
    out = jnp.maximum(v + b_ref[...].reshape(1, 1, 64), 0.0)
    o_ref[0] = out.astype(o_ref.dtype)


def _conv_fold_pool_kernel(x_ref, w_ref, b_ref, o_ref, *, cin, pool):
    _, H, W, _ = x_ref.shape
    co = o_ref.shape[-1]
    xp = jnp.pad(x_ref[0], ((1, 1), (1, 1), (0, 0)))
    acc = None
    for dy in range(3):
        rows = xp[dy:dy + H]
        patch = jnp.concatenate(
            [rows[:, 0:W], rows[:, 1:W + 1], rows[:, 2:W + 2]],
            axis=-1).reshape(H * W, 3 * cin)
        d = jnp.dot(patch, w_ref[dy], preferred_element_type=jnp.float32)
        acc = d if acc is None else acc + d
    y = acc.reshape(H, W, co)
    if pool:
        y4 = y.reshape(H // 2, 2, W, co)
        v = jnp.maximum(y4[:, 0], y4[:, 1])
        z = v.reshape(H // 2, W // 2, 2 * co)
        y = jnp.maximum(z[..., :co], z[..., co:])
    out = jnp.maximum(y + b_ref[...].reshape(1, 1, co), 0.0)
    o_ref[0] = out.astype(o_ref.dtype)


def _conv0_pool(xf, w0, b0):
    N, H, W, _ = xf.shape
    Wo = W // 2
    xp = jnp.pad(xf, ((0, 0), (1, 1), (1, 1), (0, 0)))
    xe = xp[:, :, 0::2, :]
    xo = xp[:, :, 1::2, :]
    halves = [xe, xo]
    taps_e = [halves[dx % 2][:, dy:dy + H, dx // 2:dx // 2 + Wo, :]
              for dy in range(3) for dx in range(3)]
    taps_o = [halves[(dx + 1) % 2][:, dy:dy + H,
                                   (dx + 1) // 2:(dx + 1) // 2 + Wo, :]
              for dy in range(3) for dx in range(3)]
    kpad = ((0, 0), (0, 0), (0, 0), (0, 5))
    pe = jnp.pad(jnp.concatenate(taps_e, axis=-1), kpad)
    po = jnp.pad(jnp.concatenate(taps_o, axis=-1), kpad)
    tp = 8
    return pl.pallas_call(
        _conv0_pool_kernel,
        out_shape=jax.ShapeDtypeStruct((N, H // 2, Wo, 64), jnp.bfloat16),
        grid=(N, H // (2 * tp)),
        in_specs=[
            pl.BlockSpec((1, 2 * tp, Wo, 32), lambda n, r: (n, r, 0, 0)),
            pl.BlockSpec((1, 2 * tp, Wo, 32), lambda n, r: (n, r, 0, 0)),
            pl.BlockSpec((32, 64), lambda n, r: (0, 0)),
            pl.BlockSpec((1, 64), lambda n, r: (0, 0)),
        ],
        out_specs=pl.BlockSpec((1, tp, Wo, 64), lambda n, r: (n, r, 0, 0)),
        compiler_params=pltpu.CompilerParams(
            dimension_semantics=("parallel", "parallel")),
    )(pe, po, w0, b0)


def _conv_layer(x, w, b, *, cin, cout, pool):
    N, H, W, _ = x.shape
    Ho, Wo = (H // 2, W // 2) if pool else (H, W)
    return pl.pallas_call(
        functools.partial(_conv_fold_pool_kernel, cin=cin, pool=pool),
        out_shape=jax.ShapeDtypeStruct((N, Ho, Wo, cout), jnp.bfloat16),
        grid=(N,),
        in_specs=[
            pl.BlockSpec((1, H, W, cin), lambda n: (n, 0, 0, 0)),
            pl.BlockSpec((3, 3 * cin, cout), lambda n: (0, 0, 0)),
            pl.BlockSpec((1, cout), lambda n: (0, 0)),
        ],
        out_specs=pl.BlockSpec((1, Ho, Wo, cout), lambda n: (n, 0, 0, 0)),
        compiler_params=pltpu.CompilerParams(
            dimension_semantics=("parallel",),
            vmem_limit_bytes=48 * 1024 * 1024),
    )(x, w, b)


def _fc_kernel(x_ref, w_ref, b_ref, o_ref, acc_ref, *, relu):
    k = pl.program_id(1)

    @pl.when(k == 0)
    def _init():
        acc_ref[...] = jnp.zeros_like(acc_ref)

    acc_ref[...] += jnp.dot(x_ref[...], w_ref[...],
                            preferred_element_type=jnp.float32)

    @pl.when(k == pl.num_programs(1) - 1)
    def _done():
        r = acc_ref[...] + b_ref[...]
        if relu:
            r = jnp.maximum(r, 0.0)
        o_ref[...] = r.astype(o_ref.dtype)


def _fc(x, w, b, *, relu, out_dtype):
    M, K = x.shape
    Kp, Np = w.shape
    tk = next(t for t in (4096, 3584, 2048, 1024, 512, 256, 128) if Kp % t == 0)
    tn = next(t for t in (512, 256, 128) if Np % t == 0)
    return pl.pallas_call(
        functools.partial(_fc_kernel, relu=relu),
        out_shape=jax.ShapeDtypeStruct((M, Np), out_dtype),
        grid=(Np // tn, Kp // tk),
        in_specs=[
            pl.BlockSpec((M, tk), lambda j, k: (0, k)),
            pl.BlockSpec((tk, tn), lambda j, k: (k, j)),
            pl.BlockSpec((1, tn), lambda j, k: (0, j)),
        ],
        out_specs=pl.BlockSpec((M, tn), lambda j, k: (0, j)),
        scratch_shapes=[pltpu.VMEM((M, tn), jnp.float32)],
        compiler_params=pltpu.CompilerParams(
            dimension_semantics=("parallel", "arbitrary")),
    )(x.astype(jnp.bfloat16), w, b)


def kernel(x_nchw, conv_w_0, conv_b_0, conv_w_1, conv_b_1, conv_w_2, conv_b_2,
           conv_w_3, conv_b_3, conv_w_4, conv_b_4, conv_w_5, conv_b_5,
           conv_w_6, conv_b_6, conv_w_7, conv_b_7,
           fc_w_0, fc_b_0, fc_w_1, fc_b_1, fc_w_2, fc_b_2):
    N = x_nchw.shape[0]
    xf = jnp.transpose(x_nchw.astype(jnp.bfloat16), (0, 2, 3, 1))
    h = _conv0_pool(xf, conv_w_0[:32, :64], conv_b_0[:, :64])
    w1 = conv_w_1.reshape(3, 3, 128, 128)[:, :, :64, :].reshape(3, 192, 128)
    h = _conv_layer(h, w1, conv_b_1, cin=64, cout=128, pool=True)
    h = _conv_layer(h, conv_w_2, conv_b_2, cin=128, cout=256, pool=False)
    h = _conv_layer(h, conv_w_3, conv_b_3, cin=256, cout=256, pool=True)
    h = _conv_layer(h, conv_w_4, conv_b_4, cin=256, cout=512, pool=False)
    h = _conv_layer(h, conv_w_5, conv_b_5, cin=512, cout=512, pool=True)
    h = _conv_layer(h, conv_w_6, conv_b_6, cin=512, cout=512, pool=False)
    h = _conv_layer(h, conv_w_7, conv_b_7, cin=512, cout=512, pool=True)
    f = h.reshape(N, -1)
    f = _fc(f, fc_w_0, fc_b_0, relu=True, out_dtype=jnp.bfloat16)
    f = _fc(f, fc_w_1, fc_b_1, relu=True, out_dtype=jnp.bfloat16)
    f = _fc(f, fc_w_2, fc_b_2, relu=False, out_dtype=jnp.float32)
    return f[:, :10]
